# baseline (speedup 1.0000x reference)
import functools

import jax
import jax.numpy as jnp
from jax import lax
from jax.experimental import pallas as pl
from jax.experimental.pallas import tpu as pltpu
from jax.experimental.pallas import tpu_sc as plsc

FUDGE = 1e-07

B = 320000
D = 128
NC = 2
NS = 16
L = 16
NW = NC * NS
BPW = B // NW
G = 40
NSTEPS = BPW // G
QOFFS = (0, 16, 24)


def _dots_for_chunk(s_ref, d_ref, out_ref, out_base):
    row_iota = lax.iota(jnp.int32, L)
    one = jnp.full((L,), 1.0, dtype=jnp.float32)
    scale = jnp.full((L,), 1.0 - 2.0 * FUDGE, dtype=jnp.float32)
    fudge = jnp.full((L,), FUDGE, dtype=jnp.float32)
    for q in QOFFS:
        rows = row_iota + q

        def dbody(t, acc):
            for j in range(4):
                cols = jnp.full((L,), t * 4 + j, dtype=jnp.int32)
                sv = plsc.load_gather(s_ref, [rows, cols])
                dv = plsc.load_gather(d_ref, [rows, cols])
                acc = acc + sv * dv
            return acc

        acc = lax.fori_loop(
            0, D // 4, dbody, jnp.zeros((L,), dtype=jnp.float32)
        )
        sig = one / (one + jnp.exp(-acc))
        out_ref[pl.ds(out_base + q, L)] = (sig + fudge) * scale


def _decode_kernel(z_hbm, sidx_hbm, didx_hbm, out_hbm,
                   z_sp, siA, diA, siB, diB, sA, dA, sB, dB, out_v,
                   sem_ia, sem_ib, sem_sa, sem_da, sem_sb, sem_db):
    sid = lax.axis_index("s")
    wid = sid * NC + lax.axis_index("c")

    @pl.when(sid < 10)
    def _():
        pltpu.sync_copy(z_hbm.at[pl.ds(sid * 1000, 1000)],
                        z_sp.at[pl.ds(sid * 1000, 1000)])

    def start_idx(g, si_b, di_b, sem):
        pltpu.make_async_copy(sidx_hbm.at[wid, g], si_b, sem).start()
        pltpu.make_async_copy(didx_hbm.at[wid, g], di_b, sem).start()

    def wait_idx(g, si_b, di_b, sem):
        pltpu.make_async_copy(sidx_hbm.at[wid, g], si_b, sem).wait()
        pltpu.make_async_copy(didx_hbm.at[wid, g], di_b, sem).wait()

    def start_rows(si_b, di_b, s_buf, d_buf, s_sem, d_sem):
        pltpu.make_async_copy(z_sp.at[si_b], s_buf, s_sem).start()
        pltpu.make_async_copy(z_sp.at[di_b], d_buf, d_sem).start()

    def wait_rows(si_b, di_b, s_buf, d_buf, s_sem, d_sem):
        pltpu.make_async_copy(z_sp.at[si_b], s_buf, s_sem).wait()
        pltpu.make_async_copy(z_sp.at[di_b], d_buf, d_sem).wait()

    start_idx(0, siA, diA, sem_ia)
    start_idx(1, siB, diB, sem_ib)

    plsc.subcore_barrier()

    wait_idx(0, siA, diA, sem_ia)
    start_rows(siA, diA, sA, dA, sem_sa, sem_da)

    def gbody(t, carry):
        gA = t * 2
        gB = gA + 1

        wait_rows(siA, diA, sA, dA, sem_sa, sem_da)

        @pl.when(gA + 2 < NSTEPS)
        def _():
            start_idx(gA + 2, siA, diA, sem_ia)

        wait_idx(gB, siB, diB, sem_ib)
        start_rows(siB, diB, sB, dB, sem_sb, sem_db)
        _dots_for_chunk(sA, dA, out_v, gA * G)

        wait_rows(siB, diB, sB, dB, sem_sb, sem_db)

        @pl.when(gB + 2 < NSTEPS)
        def _():
            start_idx(gB + 2, siB, diB, sem_ib)

        @pl.when(gA + 2 < NSTEPS)
        def _():
            wait_idx(gA + 2, siA, diA, sem_ia)
            start_rows(siA, diA, sA, dA, sem_sa, sem_da)

        _dots_for_chunk(sB, dB, out_v, gB * G)
        return carry

    lax.fori_loop(0, NSTEPS // 2, gbody, 0)

    pltpu.sync_copy(out_v, out_hbm.at[wid])


@jax.jit
def _decode(z, sidx, didx):
    mesh = plsc.VectorSubcoreMesh(
        core_axis_name="c", subcore_axis_name="s",
        num_cores=NC, num_subcores=NS,
    )
    f = pl.kernel(
        _decode_kernel,
        out_type=jax.ShapeDtypeStruct((NW, BPW), jnp.float32),
        mesh=mesh,
        scratch_types=[
            pltpu.VMEM_SHARED((10000, D), jnp.float32),
            pltpu.VMEM((G,), jnp.int32),
            pltpu.VMEM((G,), jnp.int32),
            pltpu.VMEM((G,), jnp.int32),
            pltpu.VMEM((G,), jnp.int32),
            pltpu.VMEM((G, D), jnp.float32),
            pltpu.VMEM((G, D), jnp.float32),
            pltpu.VMEM((G, D), jnp.float32),
            pltpu.VMEM((G, D), jnp.float32),
            pltpu.VMEM((BPW,), jnp.float32),
            pltpu.SemaphoreType.DMA,
            pltpu.SemaphoreType.DMA,
            pltpu.SemaphoreType.DMA,
            pltpu.SemaphoreType.DMA,
            pltpu.SemaphoreType.DMA,
            pltpu.SemaphoreType.DMA,
        ],
        compiler_params=pltpu.CompilerParams(needs_layout_passes=False),
    )
    return f(z, sidx, didx)


def kernel(z, rand_inds):
    ri = rand_inds.astype(jnp.int32)
    sidx = ri[0].reshape(NW, NSTEPS, G)
    didx = ri[1].reshape(NW, NSTEPS, G)
    out = _decode(z, sidx, didx)
    return out.reshape(B)


# speedup vs baseline: 7.1065x; 7.1065x over previous
import functools

import jax
import jax.numpy as jnp
from jax import lax
from jax.experimental import pallas as pl
from jax.experimental.pallas import tpu as pltpu
from jax.experimental.pallas import tpu_sc as plsc

FUDGE = 1e-07

B = 320000
D = 128
NC = 2
NS = 16
L = 16
NW = NC * NS
BPW = B // NW
G = 40
NSTEPS = BPW // G
QOFFS = (0, 16, 24)


def _dots_for_chunk(s_ref, d_ref, out_ref, out_base):
    row_iota = lax.iota(jnp.int32, L)
    one = jnp.full((L,), 1.0, dtype=jnp.float32)
    scale = jnp.full((L,), 1.0 - 2.0 * FUDGE, dtype=jnp.float32)
    fudge = jnp.full((L,), FUDGE, dtype=jnp.float32)
    for q in QOFFS:
        rows = row_iota + q

        def dbody(t, carry):
            acc, cols = carry
            for _ in range(4):
                sv = plsc.load_gather(s_ref, [rows, cols])
                dv = plsc.load_gather(d_ref, [rows, cols])
                acc = acc + sv * dv
                cols = jnp.bitwise_and(cols + 1, D - 1)
            return acc, cols

        acc, _ = lax.fori_loop(
            0, D // 4, dbody,
            (jnp.zeros((L,), dtype=jnp.float32), row_iota),
        )
        sig = one / (one + jnp.exp(-acc))
        out_ref[pl.ds(out_base + q, L)] = (sig + fudge) * scale


def _decode_kernel(z_hbm, sidx_hbm, didx_hbm, out_hbm,
                   z_sp, siA, diA, siB, diB, sA, dA, sB, dB, out_v,
                   sem_ia, sem_ib, sem_sa, sem_da, sem_sb, sem_db):
    sid = lax.axis_index("s")
    wid = sid * NC + lax.axis_index("c")

    @pl.when(sid < 10)
    def _():
        pltpu.sync_copy(z_hbm.at[pl.ds(sid * 1000, 1000)],
                        z_sp.at[pl.ds(sid * 1000, 1000)])

    def start_idx(g, si_b, di_b, sem):
        pltpu.make_async_copy(sidx_hbm.at[wid, g], si_b, sem).start()
        pltpu.make_async_copy(didx_hbm.at[wid, g], di_b, sem).start()

    def wait_idx(g, si_b, di_b, sem):
        pltpu.make_async_copy(sidx_hbm.at[wid, g], si_b, sem).wait()
        pltpu.make_async_copy(didx_hbm.at[wid, g], di_b, sem).wait()

    def start_rows(si_b, di_b, s_buf, d_buf, s_sem, d_sem):
        pltpu.make_async_copy(z_sp.at[si_b], s_buf, s_sem).start()
        pltpu.make_async_copy(z_sp.at[di_b], d_buf, d_sem).start()

    def wait_rows(si_b, di_b, s_buf, d_buf, s_sem, d_sem):
        pltpu.make_async_copy(z_sp.at[si_b], s_buf, s_sem).wait()
        pltpu.make_async_copy(z_sp.at[di_b], d_buf, d_sem).wait()

    start_idx(0, siA, diA, sem_ia)
    start_idx(1, siB, diB, sem_ib)

    plsc.subcore_barrier()

    wait_idx(0, siA, diA, sem_ia)
    start_rows(siA, diA, sA, dA, sem_sa, sem_da)

    def gbody(t, carry):
        gA = t * 2
        gB = gA + 1

        wait_rows(siA, diA, sA, dA, sem_sa, sem_da)

        @pl.when(gA + 2 < NSTEPS)
        def _():
            start_idx(gA + 2, siA, diA, sem_ia)

        wait_idx(gB, siB, diB, sem_ib)
        start_rows(siB, diB, sB, dB, sem_sb, sem_db)
        _dots_for_chunk(sA, dA, out_v, gA * G)

        wait_rows(siB, diB, sB, dB, sem_sb, sem_db)

        @pl.when(gB + 2 < NSTEPS)
        def _():
            start_idx(gB + 2, siB, diB, sem_ib)

        @pl.when(gA + 2 < NSTEPS)
        def _():
            wait_idx(gA + 2, siA, diA, sem_ia)
            start_rows(siA, diA, sA, dA, sem_sa, sem_da)

        _dots_for_chunk(sB, dB, out_v, gB * G)
        return carry

    lax.fori_loop(0, NSTEPS // 2, gbody, 0)

    pltpu.sync_copy(out_v, out_hbm.at[wid])


@jax.jit
def _decode(z, sidx, didx):
    mesh = plsc.VectorSubcoreMesh(
        core_axis_name="c", subcore_axis_name="s",
        num_cores=NC, num_subcores=NS,
    )
    f = pl.kernel(
        _decode_kernel,
        out_type=jax.ShapeDtypeStruct((NW, BPW), jnp.float32),
        mesh=mesh,
        scratch_types=[
            pltpu.VMEM_SHARED((10000, D), jnp.float32),
            pltpu.VMEM((G,), jnp.int32),
            pltpu.VMEM((G,), jnp.int32),
            pltpu.VMEM((G,), jnp.int32),
            pltpu.VMEM((G,), jnp.int32),
            pltpu.VMEM((G, D), jnp.float32),
            pltpu.VMEM((G, D), jnp.float32),
            pltpu.VMEM((G, D), jnp.float32),
            pltpu.VMEM((G, D), jnp.float32),
            pltpu.VMEM((BPW,), jnp.float32),
            pltpu.SemaphoreType.DMA,
            pltpu.SemaphoreType.DMA,
            pltpu.SemaphoreType.DMA,
            pltpu.SemaphoreType.DMA,
            pltpu.SemaphoreType.DMA,
            pltpu.SemaphoreType.DMA,
        ],
        compiler_params=pltpu.CompilerParams(needs_layout_passes=False),
    )
    return f(z, sidx, didx)


def kernel(z, rand_inds):
    ri = rand_inds.astype(jnp.int32)
    sidx = ri[0].reshape(NW, NSTEPS, G)
    didx = ri[1].reshape(NW, NSTEPS, G)
    out = _decode(z, sidx, didx)
    return out.reshape(B)


# speedup vs baseline: 7.6567x; 1.0774x over previous
import functools

import jax
import jax.numpy as jnp
from jax import lax
from jax.experimental import pallas as pl
from jax.experimental.pallas import tpu as pltpu
from jax.experimental.pallas import tpu_sc as plsc

FUDGE = 1e-07

B = 320000
D = 128
DW = D // 2
NC = 2
NS = 16
L = 16
NW = NC * NS
BPW = B // NW
G = 40
NSTEPS = BPW // G
QOFFS = (0, 16, 24)


def _dots_for_chunk(s_ref, d_ref, out_ref, out_base):
    row_iota = lax.iota(jnp.int32, L)
    one = jnp.full((L,), 1.0, dtype=jnp.float32)
    scale = jnp.full((L,), 1.0 - 2.0 * FUDGE, dtype=jnp.float32)
    fudge = jnp.full((L,), FUDGE, dtype=jnp.float32)
    for q in QOFFS:
        rows = row_iota + q

        def dbody(t, carry):
            acc, cols = carry
            for _ in range(4):
                sv = plsc.load_gather(s_ref, [rows, cols])
                dv = plsc.load_gather(d_ref, [rows, cols])
                s0, s1 = plsc.unpack(
                    plsc.bitcast(sv, jnp.bfloat16),
                    format=plsc.PackFormat.INTERLEAVED,
                )
                d0, d1 = plsc.unpack(
                    plsc.bitcast(dv, jnp.bfloat16),
                    format=plsc.PackFormat.INTERLEAVED,
                )
                acc = acc + s0 * d0 + s1 * d1
                cols = jnp.bitwise_and(cols + 1, DW - 1)
            return acc, cols

        acc, _ = lax.fori_loop(
            0, DW // 4, dbody,
            (jnp.zeros((L,), dtype=jnp.float32), row_iota),
        )
        sig = one / (one + jnp.exp(-acc))
        out_ref[pl.ds(out_base + q, L)] = (sig + fudge) * scale


def _decode_kernel(z_hbm, sidx_hbm, didx_hbm, out_hbm,
                   z_sp, siA, diA, siB, diB, sA, dA, sB, dB, out_v,
                   sem_ia, sem_ib, sem_sa, sem_da, sem_sb, sem_db):
    sid = lax.axis_index("s")
    wid = sid * NC + lax.axis_index("c")

    @pl.when(sid < 10)
    def _():
        pltpu.sync_copy(z_hbm.at[pl.ds(sid * 1000, 1000)],
                        z_sp.at[pl.ds(sid * 1000, 1000)])

    def start_idx(g, si_b, di_b, sem):
        pltpu.make_async_copy(sidx_hbm.at[wid, g], si_b, sem).start()
        pltpu.make_async_copy(didx_hbm.at[wid, g], di_b, sem).start()

    def wait_idx(g, si_b, di_b, sem):
        pltpu.make_async_copy(sidx_hbm.at[wid, g], si_b, sem).wait()
        pltpu.make_async_copy(didx_hbm.at[wid, g], di_b, sem).wait()

    def start_rows(si_b, di_b, s_buf, d_buf, s_sem, d_sem):
        pltpu.make_async_copy(z_sp.at[si_b], s_buf, s_sem).start()
        pltpu.make_async_copy(z_sp.at[di_b], d_buf, d_sem).start()

    def wait_rows(si_b, di_b, s_buf, d_buf, s_sem, d_sem):
        pltpu.make_async_copy(z_sp.at[si_b], s_buf, s_sem).wait()
        pltpu.make_async_copy(z_sp.at[di_b], d_buf, d_sem).wait()

    start_idx(0, siA, diA, sem_ia)
    start_idx(1, siB, diB, sem_ib)

    plsc.subcore_barrier()

    wait_idx(0, siA, diA, sem_ia)
    start_rows(siA, diA, sA, dA, sem_sa, sem_da)

    def gbody(t, carry):
        gA = t * 2
        gB = gA + 1

        wait_rows(siA, diA, sA, dA, sem_sa, sem_da)

        @pl.when(gA + 2 < NSTEPS)
        def _():
            start_idx(gA + 2, siA, diA, sem_ia)

        wait_idx(gB, siB, diB, sem_ib)
        start_rows(siB, diB, sB, dB, sem_sb, sem_db)
        _dots_for_chunk(sA, dA, out_v, gA * G)

        wait_rows(siB, diB, sB, dB, sem_sb, sem_db)

        @pl.when(gB + 2 < NSTEPS)
        def _():
            start_idx(gB + 2, siB, diB, sem_ib)

        @pl.when(gA + 2 < NSTEPS)
        def _():
            wait_idx(gA + 2, siA, diA, sem_ia)
            start_rows(siA, diA, sA, dA, sem_sa, sem_da)

        _dots_for_chunk(sB, dB, out_v, gB * G)
        return carry

    lax.fori_loop(0, NSTEPS // 2, gbody, 0)

    pltpu.sync_copy(out_v, out_hbm.at[wid])


@jax.jit
def _decode(z32, sidx, didx):
    mesh = plsc.VectorSubcoreMesh(
        core_axis_name="c", subcore_axis_name="s",
        num_cores=NC, num_subcores=NS,
    )
    f = pl.kernel(
        _decode_kernel,
        out_type=jax.ShapeDtypeStruct((NW, BPW), jnp.float32),
        mesh=mesh,
        scratch_types=[
            pltpu.VMEM_SHARED((10000, DW), jnp.int32),
            pltpu.VMEM((G,), jnp.int32),
            pltpu.VMEM((G,), jnp.int32),
            pltpu.VMEM((G,), jnp.int32),
            pltpu.VMEM((G,), jnp.int32),
            pltpu.VMEM((G, DW), jnp.int32),
            pltpu.VMEM((G, DW), jnp.int32),
            pltpu.VMEM((G, DW), jnp.int32),
            pltpu.VMEM((G, DW), jnp.int32),
            pltpu.VMEM((BPW,), jnp.float32),
            pltpu.SemaphoreType.DMA,
            pltpu.SemaphoreType.DMA,
            pltpu.SemaphoreType.DMA,
            pltpu.SemaphoreType.DMA,
            pltpu.SemaphoreType.DMA,
            pltpu.SemaphoreType.DMA,
        ],
        compiler_params=pltpu.CompilerParams(
            needs_layout_passes=False, use_tc_tiling_on_sc=False,
        ),
    )
    return f(z32, sidx, didx)


def kernel(z, rand_inds):
    z32 = lax.bitcast_convert_type(
        z.astype(jnp.bfloat16).reshape(10000, DW, 2), jnp.int32
    )
    ri = rand_inds.astype(jnp.int32)
    sidx = ri[0].reshape(NW, NSTEPS, G)
    didx = ri[1].reshape(NW, NSTEPS, G)
    out = _decode(z32, sidx, didx)
    return out.reshape(B)


# speedup vs baseline: 8.6330x; 1.1275x over previous
import functools

import jax
import jax.numpy as jnp
from jax import lax
from jax.experimental import pallas as pl
from jax.experimental.pallas import tpu as pltpu
from jax.experimental.pallas import tpu_sc as plsc

FUDGE = 1e-07

B = 320000
D = 128
DW = D // 2
NC = 2
NS = 16
L = 16
NW = NC * NS
BPW = B // NW
G = 80
NSTEPS = BPW // G
QOFFS = (0, 16, 32, 48, 64)


def _dots_for_chunk(s_ref, d_ref, out_ref, out_base):
    row_iota = lax.iota(jnp.int32, L)
    one = jnp.full((L,), 1.0, dtype=jnp.float32)
    scale = jnp.full((L,), 1.0 - 2.0 * FUDGE, dtype=jnp.float32)
    fudge = jnp.full((L,), FUDGE, dtype=jnp.float32)
    for q in QOFFS:
        rows = row_iota + q

        def dbody(t, carry):
            acc, cols = carry
            for _ in range(4):
                sv = plsc.load_gather(s_ref, [rows, cols])
                dv = plsc.load_gather(d_ref, [rows, cols])
                s0, s1 = plsc.unpack(
                    plsc.bitcast(sv, jnp.bfloat16),
                    format=plsc.PackFormat.INTERLEAVED,
                )
                d0, d1 = plsc.unpack(
                    plsc.bitcast(dv, jnp.bfloat16),
                    format=plsc.PackFormat.INTERLEAVED,
                )
                acc = acc + s0 * d0 + s1 * d1
                cols = jnp.bitwise_and(cols + 1, DW - 1)
            return acc, cols

        acc, _ = lax.fori_loop(
            0, DW // 4, dbody,
            (jnp.zeros((L,), dtype=jnp.float32), row_iota),
        )
        sig = one / (one + jnp.exp(-acc))
        out_ref[pl.ds(out_base + q, L)] = (sig + fudge) * scale


def _decode_kernel(z_hbm, sidx_hbm, didx_hbm, out_hbm,
                   z_sp, siA, diA, siB, diB, sA, dA, sB, dB, out_v,
                   sem_ia, sem_ib, sem_sa, sem_da, sem_sb, sem_db):
    sid = lax.axis_index("s")
    wid = sid * NC + lax.axis_index("c")

    @pl.when(sid < 10)
    def _():
        pltpu.sync_copy(z_hbm.at[pl.ds(sid * 1000, 1000)],
                        z_sp.at[pl.ds(sid * 1000, 1000)])

    def start_idx(g, si_b, di_b, sem):
        pltpu.make_async_copy(sidx_hbm.at[wid, g], si_b, sem).start()
        pltpu.make_async_copy(didx_hbm.at[wid, g], di_b, sem).start()

    def wait_idx(g, si_b, di_b, sem):
        pltpu.make_async_copy(sidx_hbm.at[wid, g], si_b, sem).wait()
        pltpu.make_async_copy(didx_hbm.at[wid, g], di_b, sem).wait()

    def start_rows(si_b, di_b, s_buf, d_buf, s_sem, d_sem):
        pltpu.make_async_copy(z_sp.at[si_b], s_buf, s_sem).start()
        pltpu.make_async_copy(z_sp.at[di_b], d_buf, d_sem).start()

    def wait_rows(si_b, di_b, s_buf, d_buf, s_sem, d_sem):
        pltpu.make_async_copy(z_sp.at[si_b], s_buf, s_sem).wait()
        pltpu.make_async_copy(z_sp.at[di_b], d_buf, d_sem).wait()

    start_idx(0, siA, diA, sem_ia)
    start_idx(1, siB, diB, sem_ib)

    plsc.subcore_barrier()

    wait_idx(0, siA, diA, sem_ia)
    start_rows(siA, diA, sA, dA, sem_sa, sem_da)

    def gbody(t, carry):
        gA = t * 2
        gB = gA + 1

        wait_rows(siA, diA, sA, dA, sem_sa, sem_da)

        @pl.when(gA + 2 < NSTEPS)
        def _():
            start_idx(gA + 2, siA, diA, sem_ia)

        wait_idx(gB, siB, diB, sem_ib)
        start_rows(siB, diB, sB, dB, sem_sb, sem_db)
        _dots_for_chunk(sA, dA, out_v, gA * G)

        wait_rows(siB, diB, sB, dB, sem_sb, sem_db)

        @pl.when(gB + 2 < NSTEPS)
        def _():
            start_idx(gB + 2, siB, diB, sem_ib)

        @pl.when(gA + 2 < NSTEPS)
        def _():
            wait_idx(gA + 2, siA, diA, sem_ia)
            start_rows(siA, diA, sA, dA, sem_sa, sem_da)

        _dots_for_chunk(sB, dB, out_v, gB * G)
        return carry

    lax.fori_loop(0, NSTEPS // 2, gbody, 0)

    wait_rows(siA, diA, sA, dA, sem_sa, sem_da)
    _dots_for_chunk(sA, dA, out_v, (NSTEPS - 1) * G)

    pltpu.sync_copy(out_v, out_hbm.at[wid])


@jax.jit
def _decode(z32, sidx, didx):
    mesh = plsc.VectorSubcoreMesh(
        core_axis_name="c", subcore_axis_name="s",
        num_cores=NC, num_subcores=NS,
    )
    f = pl.kernel(
        _decode_kernel,
        out_type=jax.ShapeDtypeStruct((NW, BPW), jnp.float32),
        mesh=mesh,
        scratch_types=[
            pltpu.VMEM_SHARED((10000, DW), jnp.int32),
            pltpu.VMEM((G,), jnp.int32),
            pltpu.VMEM((G,), jnp.int32),
            pltpu.VMEM((G,), jnp.int32),
            pltpu.VMEM((G,), jnp.int32),
            pltpu.VMEM((G, DW), jnp.int32),
            pltpu.VMEM((G, DW), jnp.int32),
            pltpu.VMEM((G, DW), jnp.int32),
            pltpu.VMEM((G, DW), jnp.int32),
            pltpu.VMEM((BPW,), jnp.float32),
            pltpu.SemaphoreType.DMA,
            pltpu.SemaphoreType.DMA,
            pltpu.SemaphoreType.DMA,
            pltpu.SemaphoreType.DMA,
            pltpu.SemaphoreType.DMA,
            pltpu.SemaphoreType.DMA,
        ],
        compiler_params=pltpu.CompilerParams(
            needs_layout_passes=False, use_tc_tiling_on_sc=False,
        ),
    )
    return f(z32, sidx, didx)


def kernel(z, rand_inds):
    z32 = lax.bitcast_convert_type(
        z.astype(jnp.bfloat16).reshape(10000, DW, 2), jnp.int32
    )
    ri = rand_inds.astype(jnp.int32)
    sidx = ri[0].reshape(NW, NSTEPS, G)
    didx = ri[1].reshape(NW, NSTEPS, G)
    out = _decode(z32, sidx, didx)
    return out.reshape(B)


# speedup vs baseline: 10.2566x; 1.1881x over previous
import functools

import jax
import jax.numpy as jnp
from jax import lax
from jax.experimental import pallas as pl
from jax.experimental.pallas import tpu as pltpu
from jax.experimental.pallas import tpu_sc as plsc

FUDGE = 1e-07

B = 320000
D = 128
DW = D // 2
NC = 2
NS = 16
L = 16
NW = NC * NS
BPW = B // NW
G = 80
NSTEPS = BPW // G
QOFFS = (0, 16, 32, 48, 64)


def _dots_for_chunk(s_ref, d_ref, out_ref, out_base):
    row_iota = lax.iota(jnp.int32, L)
    one = jnp.full((L,), 1.0, dtype=jnp.float32)
    scale = jnp.full((L,), 1.0 - 2.0 * FUDGE, dtype=jnp.float32)
    fudge = jnp.full((L,), FUDGE, dtype=jnp.float32)
    for q in QOFFS:
        rows = row_iota + q

        def dbody(t, carry):
            acc0, acc1, cols = carry
            for _ in range(4):
                sv = plsc.load_gather(s_ref, [rows, cols])
                dv = plsc.load_gather(d_ref, [rows, cols])
                p = plsc.bitcast(sv, jnp.bfloat16) * plsc.bitcast(
                    dv, jnp.bfloat16)
                p0, p1 = plsc.unpack(p, format=plsc.PackFormat.INTERLEAVED)
                acc0 = acc0 + p0
                acc1 = acc1 + p1
                cols = jnp.bitwise_and(cols + 1, DW - 1)
            return acc0, acc1, cols

        acc0, acc1, _ = lax.fori_loop(
            0, DW // 4, dbody,
            (jnp.zeros((L,), dtype=jnp.float32),
             jnp.zeros((L,), dtype=jnp.float32), row_iota),
        )
        acc = acc0 + acc1
        sig = one / (one + jnp.exp(-acc))
        out_ref[pl.ds(out_base + q, L)] = (sig + fudge) * scale


def _decode_kernel(z_hbm, sidx_hbm, didx_hbm, out_hbm,
                   z_sp, siA, diA, siB, diB, sA, dA, sB, dB, out_v,
                   sem_ia, sem_ib, sem_sa, sem_da, sem_sb, sem_db):
    sid = lax.axis_index("s")
    wid = sid * NC + lax.axis_index("c")

    @pl.when(sid < 10)
    def _():
        pltpu.sync_copy(z_hbm.at[pl.ds(sid * 1000, 1000)],
                        z_sp.at[pl.ds(sid * 1000, 1000)])

    def start_idx(g, si_b, di_b, sem):
        pltpu.make_async_copy(sidx_hbm.at[wid, g], si_b, sem).start()
        pltpu.make_async_copy(didx_hbm.at[wid, g], di_b, sem).start()

    def wait_idx(g, si_b, di_b, sem):
        pltpu.make_async_copy(sidx_hbm.at[wid, g], si_b, sem).wait()
        pltpu.make_async_copy(didx_hbm.at[wid, g], di_b, sem).wait()

    def start_rows(si_b, di_b, s_buf, d_buf, s_sem, d_sem):
        pltpu.make_async_copy(z_sp.at[si_b], s_buf, s_sem).start()
        pltpu.make_async_copy(z_sp.at[di_b], d_buf, d_sem).start()

    def wait_rows(si_b, di_b, s_buf, d_buf, s_sem, d_sem):
        pltpu.make_async_copy(z_sp.at[si_b], s_buf, s_sem).wait()
        pltpu.make_async_copy(z_sp.at[di_b], d_buf, d_sem).wait()

    start_idx(0, siA, diA, sem_ia)
    start_idx(1, siB, diB, sem_ib)

    plsc.subcore_barrier()

    wait_idx(0, siA, diA, sem_ia)
    start_rows(siA, diA, sA, dA, sem_sa, sem_da)

    def gbody(t, carry):
        gA = t * 2
        gB = gA + 1

        wait_rows(siA, diA, sA, dA, sem_sa, sem_da)

        @pl.when(gA + 2 < NSTEPS)
        def _():
            start_idx(gA + 2, siA, diA, sem_ia)

        wait_idx(gB, siB, diB, sem_ib)
        start_rows(siB, diB, sB, dB, sem_sb, sem_db)
        _dots_for_chunk(sA, dA, out_v, gA * G)

        wait_rows(siB, diB, sB, dB, sem_sb, sem_db)

        @pl.when(gB + 2 < NSTEPS)
        def _():
            start_idx(gB + 2, siB, diB, sem_ib)

        @pl.when(gA + 2 < NSTEPS)
        def _():
            wait_idx(gA + 2, siA, diA, sem_ia)
            start_rows(siA, diA, sA, dA, sem_sa, sem_da)

        _dots_for_chunk(sB, dB, out_v, gB * G)
        return carry

    lax.fori_loop(0, NSTEPS // 2, gbody, 0)

    wait_rows(siA, diA, sA, dA, sem_sa, sem_da)
    _dots_for_chunk(sA, dA, out_v, (NSTEPS - 1) * G)

    pltpu.sync_copy(out_v, out_hbm.at[wid])


@jax.jit
def _decode(z32, sidx, didx):
    mesh = plsc.VectorSubcoreMesh(
        core_axis_name="c", subcore_axis_name="s",
        num_cores=NC, num_subcores=NS,
    )
    f = pl.kernel(
        _decode_kernel,
        out_type=jax.ShapeDtypeStruct((NW, BPW), jnp.float32),
        mesh=mesh,
        scratch_types=[
            pltpu.VMEM_SHARED((10000, DW), jnp.int32),
            pltpu.VMEM((G,), jnp.int32),
            pltpu.VMEM((G,), jnp.int32),
            pltpu.VMEM((G,), jnp.int32),
            pltpu.VMEM((G,), jnp.int32),
            pltpu.VMEM((G, DW), jnp.int32),
            pltpu.VMEM((G, DW), jnp.int32),
            pltpu.VMEM((G, DW), jnp.int32),
            pltpu.VMEM((G, DW), jnp.int32),
            pltpu.VMEM((BPW,), jnp.float32),
            pltpu.SemaphoreType.DMA,
            pltpu.SemaphoreType.DMA,
            pltpu.SemaphoreType.DMA,
            pltpu.SemaphoreType.DMA,
            pltpu.SemaphoreType.DMA,
            pltpu.SemaphoreType.DMA,
        ],
        compiler_params=pltpu.CompilerParams(
            needs_layout_passes=False, use_tc_tiling_on_sc=False,
        ),
    )
    return f(z32, sidx, didx)


def kernel(z, rand_inds):
    z32 = lax.bitcast_convert_type(
        z.astype(jnp.bfloat16).reshape(10000, DW, 2), jnp.int32
    )
    ri = rand_inds.astype(jnp.int32)
    sidx = ri[0].reshape(NW, NSTEPS, G)
    didx = ri[1].reshape(NW, NSTEPS, G)
    out = _decode(z32, sidx, didx)
    return out.reshape(B)


# speedup vs baseline: 10.2753x; 1.0018x over previous
import functools

import jax
import jax.numpy as jnp
from jax import lax
from jax.experimental import pallas as pl
from jax.experimental.pallas import tpu as pltpu
from jax.experimental.pallas import tpu_sc as plsc

FUDGE = 1e-07

B = 320000
D = 128
DW = D // 2
NC = 2
NS = 16
L = 16
NW = NC * NS
BPW = B // NW
G = 80
NSTEPS = BPW // G
QOFFS = (0, 16, 32, 48, 64)


def _dots_for_chunk(s_ref, d_ref, out_ref, out_base):
    row_iota = lax.iota(jnp.int32, L)
    one = jnp.full((L,), 1.0, dtype=jnp.float32)
    scale = jnp.full((L,), 1.0 - 2.0 * FUDGE, dtype=jnp.float32)
    fudge = jnp.full((L,), FUDGE, dtype=jnp.float32)
    for q in QOFFS:
        rows = row_iota + q

        def dbody(t, carry):
            acc0, acc1, cols = carry
            for _ in range(8):
                sv = plsc.load_gather(s_ref, [rows, cols])
                dv = plsc.load_gather(d_ref, [rows, cols])
                p = plsc.bitcast(sv, jnp.bfloat16) * plsc.bitcast(
                    dv, jnp.bfloat16)
                p0, p1 = plsc.unpack(p, format=plsc.PackFormat.INTERLEAVED)
                acc0 = acc0 + p0
                acc1 = acc1 + p1
                cols = jnp.bitwise_and(cols + 1, DW - 1)
            return acc0, acc1, cols

        acc0, acc1, _ = lax.fori_loop(
            0, DW // 8, dbody,
            (jnp.zeros((L,), dtype=jnp.float32),
             jnp.zeros((L,), dtype=jnp.float32), row_iota),
        )
        acc = acc0 + acc1
        sig = one / (one + jnp.exp(-acc))
        out_ref[pl.ds(out_base + q, L)] = (sig + fudge) * scale


def _decode_kernel(z_hbm, sidx_hbm, didx_hbm, out_hbm,
                   z_sp, siA, diA, siB, diB, sA, dA, sB, dB, out_v,
                   sem_ia, sem_ib, sem_sa, sem_da, sem_sb, sem_db):
    sid = lax.axis_index("s")
    wid = sid * NC + lax.axis_index("c")

    @pl.when(sid < 10)
    def _():
        pltpu.sync_copy(z_hbm.at[pl.ds(sid * 1000, 1000)],
                        z_sp.at[pl.ds(sid * 1000, 1000)])

    def start_idx(g, si_b, di_b, sem):
        pltpu.make_async_copy(sidx_hbm.at[wid, g], si_b, sem).start()
        pltpu.make_async_copy(didx_hbm.at[wid, g], di_b, sem).start()

    def wait_idx(g, si_b, di_b, sem):
        pltpu.make_async_copy(sidx_hbm.at[wid, g], si_b, sem).wait()
        pltpu.make_async_copy(didx_hbm.at[wid, g], di_b, sem).wait()

    def start_rows(si_b, di_b, s_buf, d_buf, s_sem, d_sem):
        pltpu.make_async_copy(z_sp.at[si_b], s_buf, s_sem).start()
        pltpu.make_async_copy(z_sp.at[di_b], d_buf, d_sem).start()

    def wait_rows(si_b, di_b, s_buf, d_buf, s_sem, d_sem):
        pltpu.make_async_copy(z_sp.at[si_b], s_buf, s_sem).wait()
        pltpu.make_async_copy(z_sp.at[di_b], d_buf, d_sem).wait()

    start_idx(0, siA, diA, sem_ia)
    start_idx(1, siB, diB, sem_ib)

    plsc.subcore_barrier()

    wait_idx(0, siA, diA, sem_ia)
    start_rows(siA, diA, sA, dA, sem_sa, sem_da)

    def gbody(t, carry):
        gA = t * 2
        gB = gA + 1

        wait_rows(siA, diA, sA, dA, sem_sa, sem_da)

        @pl.when(gA + 2 < NSTEPS)
        def _():
            start_idx(gA + 2, siA, diA, sem_ia)

        wait_idx(gB, siB, diB, sem_ib)
        start_rows(siB, diB, sB, dB, sem_sb, sem_db)
        _dots_for_chunk(sA, dA, out_v, gA * G)

        wait_rows(siB, diB, sB, dB, sem_sb, sem_db)

        @pl.when(gB + 2 < NSTEPS)
        def _():
            start_idx(gB + 2, siB, diB, sem_ib)

        @pl.when(gA + 2 < NSTEPS)
        def _():
            wait_idx(gA + 2, siA, diA, sem_ia)
            start_rows(siA, diA, sA, dA, sem_sa, sem_da)

        _dots_for_chunk(sB, dB, out_v, gB * G)
        return carry

    lax.fori_loop(0, NSTEPS // 2, gbody, 0)

    wait_rows(siA, diA, sA, dA, sem_sa, sem_da)
    _dots_for_chunk(sA, dA, out_v, (NSTEPS - 1) * G)

    pltpu.sync_copy(out_v, out_hbm.at[wid])


@jax.jit
def _decode(z32, sidx, didx):
    mesh = plsc.VectorSubcoreMesh(
        core_axis_name="c", subcore_axis_name="s",
        num_cores=NC, num_subcores=NS,
    )
    f = pl.kernel(
        _decode_kernel,
        out_type=jax.ShapeDtypeStruct((NW, BPW), jnp.float32),
        mesh=mesh,
        scratch_types=[
            pltpu.VMEM_SHARED((10000, DW), jnp.int32),
            pltpu.VMEM((G,), jnp.int32),
            pltpu.VMEM((G,), jnp.int32),
            pltpu.VMEM((G,), jnp.int32),
            pltpu.VMEM((G,), jnp.int32),
            pltpu.VMEM((G, DW), jnp.int32),
            pltpu.VMEM((G, DW), jnp.int32),
            pltpu.VMEM((G, DW), jnp.int32),
            pltpu.VMEM((G, DW), jnp.int32),
            pltpu.VMEM((BPW,), jnp.float32),
            pltpu.SemaphoreType.DMA,
            pltpu.SemaphoreType.DMA,
            pltpu.SemaphoreType.DMA,
            pltpu.SemaphoreType.DMA,
            pltpu.SemaphoreType.DMA,
            pltpu.SemaphoreType.DMA,
        ],
        compiler_params=pltpu.CompilerParams(
            needs_layout_passes=False, use_tc_tiling_on_sc=False,
        ),
    )
    return f(z32, sidx, didx)


def kernel(z, rand_inds):
    z32 = lax.bitcast_convert_type(
        z.astype(jnp.bfloat16).reshape(10000, DW, 2), jnp.int32
    )
    ri = rand_inds.astype(jnp.int32)
    sidx = ri[0].reshape(NW, NSTEPS, G)
    didx = ri[1].reshape(NW, NSTEPS, G)
    out = _decode(z32, sidx, didx)
    return out.reshape(B)
